# Initial kernel scaffold; baseline (speedup 1.0000x reference)
import jax
import jax.numpy as jnp
from jax.experimental import pallas as pl


def kernel(inputs, W, b):
    raise NotImplementedError("write your pallas kernel here")


# speedup vs baseline: 5.4632x; 5.4632x over previous
import jax
import jax.numpy as jnp
from jax.experimental import pallas as pl

N_TOKENS = 16384
EMBED = 2048
N_EXPERTS = 64
TOP_K = 8
BLK = 512


def _router_kernel(x_ref, w_ref, b_ref, probs_ref, idx_ref):
    x = x_ref[...]
    w = w_ref[...]
    logits = jax.lax.dot_general(
        x, w, (((1,), (1,)), ((), ())),
        preferred_element_type=jnp.float32,
        precision=jax.lax.Precision.DEFAULT,
    ) + b_ref[...]

    iota = jax.lax.broadcasted_iota(jnp.int32, logits.shape, 1)
    selected = jnp.zeros(logits.shape, jnp.bool_)
    neg = jnp.float32(-jnp.inf)
    cur = logits
    idx_cols = []
    for _ in range(TOP_K):
        m = jnp.max(cur, axis=1, keepdims=True)
        idx = jnp.min(jnp.where(cur == m, iota, N_EXPERTS), axis=1, keepdims=True)
        onehot = iota == idx
        selected = jnp.logical_or(selected, onehot)
        cur = jnp.where(onehot, neg, cur)
        idx_cols.append(idx)
    idx_ref[...] = jnp.concatenate(idx_cols, axis=1)

    mx = jnp.max(jnp.where(selected, logits, neg), axis=1, keepdims=True)
    e = jnp.where(selected, jnp.exp(logits - mx), 0.0)
    probs_ref[...] = e / jnp.sum(e, axis=1, keepdims=True)


@jax.jit
def kernel(inputs, W, b):
    b2 = b.reshape(1, N_EXPERTS)
    probs, idx = pl.pallas_call(
        _router_kernel,
        grid=(N_TOKENS // BLK,),
        in_specs=[
            pl.BlockSpec((BLK, EMBED), lambda i: (i, 0)),
            pl.BlockSpec((N_EXPERTS, EMBED), lambda i: (0, 0)),
            pl.BlockSpec((1, N_EXPERTS), lambda i: (0, 0)),
        ],
        out_specs=[
            pl.BlockSpec((BLK, N_EXPERTS), lambda i: (i, 0)),
            pl.BlockSpec((BLK, TOP_K), lambda i: (i, 0)),
        ],
        out_shape=[
            jax.ShapeDtypeStruct((N_TOKENS, N_EXPERTS), jnp.float32),
            jax.ShapeDtypeStruct((N_TOKENS, TOP_K), jnp.int32),
        ],
    )(inputs, W, b2)
    return (probs, idx)


# speedup vs baseline: 7.9386x; 1.4531x over previous
import jax
import jax.numpy as jnp
from jax.experimental import pallas as pl

N_TOKENS = 16384
EMBED = 2048
N_EXPERTS = 64
TOP_K = 8
BLK = 512


def _router_kernel(x_ref, w_ref, b_ref, probs_ref, idx_ref):
    x = x_ref[...]
    w = w_ref[...]
    lt = jax.lax.dot_general(
        w, x, (((1,), (1,)), ((), ())),
        preferred_element_type=jnp.float32,
        precision=jax.lax.Precision.DEFAULT,
    ) + b_ref[...]

    iota0 = jax.lax.broadcasted_iota(jnp.int32, lt.shape, 0)
    neg = jnp.float32(-jnp.inf)
    cur = lt
    idx_rows = []
    for _ in range(TOP_K):
        m = jnp.max(cur, axis=0, keepdims=True)
        idx = jnp.min(jnp.where(cur == m, iota0, N_EXPERTS), axis=0, keepdims=True)
        cur = jnp.where(iota0 == idx, neg, cur)
        idx_rows.append(idx)
    idx_ref[...] = jnp.concatenate(idx_rows, axis=0).T

    selected = cur == neg
    mx = jnp.max(jnp.where(selected, lt, neg), axis=0, keepdims=True)
    e = jnp.where(selected, jnp.exp(lt - mx), 0.0)
    probs_ref[...] = (e / jnp.sum(e, axis=0, keepdims=True)).T


@jax.jit
def kernel(inputs, W, b):
    b2 = b.reshape(N_EXPERTS, 1)
    probs, idx = pl.pallas_call(
        _router_kernel,
        grid=(N_TOKENS // BLK,),
        in_specs=[
            pl.BlockSpec((BLK, EMBED), lambda i: (i, 0)),
            pl.BlockSpec((N_EXPERTS, EMBED), lambda i: (0, 0)),
            pl.BlockSpec((N_EXPERTS, 1), lambda i: (0, 0)),
        ],
        out_specs=[
            pl.BlockSpec((BLK, N_EXPERTS), lambda i: (i, 0)),
            pl.BlockSpec((BLK, TOP_K), lambda i: (i, 0)),
        ],
        out_shape=[
            jax.ShapeDtypeStruct((N_TOKENS, N_EXPERTS), jnp.float32),
            jax.ShapeDtypeStruct((N_TOKENS, TOP_K), jnp.int32),
        ],
    )(inputs, W, b2)
    return (probs, idx)


# speedup vs baseline: 9.1595x; 1.1538x over previous
import jax
import jax.numpy as jnp
from jax.experimental import pallas as pl

N_TOKENS = 16384
EMBED = 2048
N_EXPERTS = 64
TOP_K = 8
BLK = 1024


def _router_kernel(x_ref, w_ref, b_ref, probs_ref, idx_ref):
    x = x_ref[...]
    w = w_ref[...]
    lt = jax.lax.dot_general(
        w, x, (((1,), (1,)), ((), ())),
        preferred_element_type=jnp.float32,
        precision=jax.lax.Precision.DEFAULT,
    ) + b_ref[...]

    iota0 = jax.lax.broadcasted_iota(jnp.int32, lt.shape, 0)
    neg = jnp.float32(-jnp.inf)
    cur = lt
    idx_rows = []
    for _ in range(TOP_K):
        m = jnp.max(cur, axis=0, keepdims=True)
        idx = jnp.min(jnp.where(cur == m, iota0, N_EXPERTS), axis=0, keepdims=True)
        cur = jnp.where(iota0 == idx, neg, cur)
        idx_rows.append(idx)
    idx_ref[...] = jnp.concatenate(idx_rows, axis=0).T

    selected = cur == neg
    mx = jnp.max(jnp.where(selected, lt, neg), axis=0, keepdims=True)
    e = jnp.where(selected, jnp.exp(lt - mx), 0.0)
    probs_ref[...] = (e / jnp.sum(e, axis=0, keepdims=True)).T


@jax.jit
def kernel(inputs, W, b):
    b2 = b.reshape(N_EXPERTS, 1)
    probs, idx = pl.pallas_call(
        _router_kernel,
        grid=(N_TOKENS // BLK,),
        in_specs=[
            pl.BlockSpec((BLK, EMBED), lambda i: (i, 0)),
            pl.BlockSpec((N_EXPERTS, EMBED), lambda i: (0, 0)),
            pl.BlockSpec((N_EXPERTS, 1), lambda i: (0, 0)),
        ],
        out_specs=[
            pl.BlockSpec((BLK, N_EXPERTS), lambda i: (i, 0)),
            pl.BlockSpec((BLK, TOP_K), lambda i: (i, 0)),
        ],
        out_shape=[
            jax.ShapeDtypeStruct((N_TOKENS, N_EXPERTS), jnp.float32),
            jax.ShapeDtypeStruct((N_TOKENS, TOP_K), jnp.int32),
        ],
    )(inputs, W, b2)
    return (probs, idx)


# speedup vs baseline: 9.6407x; 1.0525x over previous
import jax
import jax.numpy as jnp
from jax.experimental import pallas as pl

N_TOKENS = 16384
EMBED = 2048
N_EXPERTS = 64
TOP_K = 8
BLK = 2048


def _router_kernel(x_ref, w_ref, b_ref, probs_ref, idx_ref):
    x = x_ref[...]
    w = w_ref[...]
    lt = jax.lax.dot_general(
        w, x, (((1,), (1,)), ((), ())),
        preferred_element_type=jnp.float32,
        precision=jax.lax.Precision.DEFAULT,
    ) + b_ref[...]

    iota0 = jax.lax.broadcasted_iota(jnp.int32, lt.shape, 0)
    neg = jnp.float32(-jnp.inf)
    cur = lt
    idx_rows = []
    for _ in range(TOP_K):
        m = jnp.max(cur, axis=0, keepdims=True)
        idx = jnp.min(jnp.where(cur == m, iota0, N_EXPERTS), axis=0, keepdims=True)
        cur = jnp.where(iota0 == idx, neg, cur)
        idx_rows.append(idx)
    idx_ref[...] = jnp.concatenate(idx_rows, axis=0).T

    selected = cur == neg
    mx = jnp.max(jnp.where(selected, lt, neg), axis=0, keepdims=True)
    e = jnp.where(selected, jnp.exp(lt - mx), 0.0)
    probs_ref[...] = (e / jnp.sum(e, axis=0, keepdims=True)).T


@jax.jit
def kernel(inputs, W, b):
    b2 = b.reshape(N_EXPERTS, 1)
    probs, idx = pl.pallas_call(
        _router_kernel,
        grid=(N_TOKENS // BLK,),
        in_specs=[
            pl.BlockSpec((BLK, EMBED), lambda i: (i, 0)),
            pl.BlockSpec((N_EXPERTS, EMBED), lambda i: (0, 0)),
            pl.BlockSpec((N_EXPERTS, 1), lambda i: (0, 0)),
        ],
        out_specs=[
            pl.BlockSpec((BLK, N_EXPERTS), lambda i: (i, 0)),
            pl.BlockSpec((BLK, TOP_K), lambda i: (i, 0)),
        ],
        out_shape=[
            jax.ShapeDtypeStruct((N_TOKENS, N_EXPERTS), jnp.float32),
            jax.ShapeDtypeStruct((N_TOKENS, TOP_K), jnp.int32),
        ],
    )(inputs, W, b2)
    return (probs, idx)
